# baseline (speedup 1.0000x reference)
import functools
import numpy as np
import jax
import jax.numpy as jnp
from jax import lax
from jax.experimental import pallas as pl
from jax.experimental.pallas import tpu as pltpu
from jax.experimental.pallas import tpu_sc as plsc

G = 16
M = 8
HID = 32
N_OUT = 8192
K_NBR = 16
RADIUS = 0.1
N_LAYERS = 4


def _dft_consts():
    n = np.arange(G)
    k = np.arange(M)
    F = np.exp(-2j * np.pi * np.outer(k, n) / G) / G
    K3 = np.kron(F, np.kron(F, F))
    E = np.exp(2j * np.pi * np.outer(n, k) / G)
    w = np.ones(M); w[1:] = 2.0
    B3 = np.kron(E, np.kron(E, E * w[None, :]))
    c = {
        'KrT': K3.real.T, 'KiT': K3.imag.T,
        'BrT': B3.T.real, 'BiT': B3.T.imag,
    }
    return {k_: jnp.asarray(v, jnp.float32) for k_, v in c.items()}

_C = _dft_consts()


def _lift_body(h0T, w1t, b1, w2t, b2, out_ref):
    act = jax.nn.gelu
    dot = functools.partial(jnp.dot, preferred_element_type=jnp.float32)
    h = act(dot(w1t[...], h0T[...]) + b1[...])
    out_ref[...] = dot(w2t[...], h) + b2[...]


def _fwd_body(hT, wrl, wil, krt, kit, sr_ref, si_ref):
    dot = functools.partial(jnp.dot, preferred_element_type=jnp.float32)
    h = hT[...]
    Xr = dot(h, krt[...])
    Xi = dot(h, kit[...])
    Wrl = wrl[...]
    Wil = wil[...]
    Xr3 = Xr[:, None, :]
    Xi3 = Xi[:, None, :]
    sr_ref[...] = jnp.sum(Xr3 * Wrl - Xi3 * Wil, axis=0)
    si_ref[...] = jnp.sum(Xr3 * Wil + Xi3 * Wrl, axis=0)


def _make_inv_body(last):
    def _inv_body(sr, si, hT, wst_l, bs_l, brt, bit, out_ref):
        dot = functools.partial(jnp.dot, preferred_element_type=jnp.float32)
        xs = dot(sr[...], brt[...]) - dot(si[...], bit[...])
        h = xs + dot(wst_l[...], hT[...]) + bs_l[...]
        if last:
            out_ref[...] = h.T
        else:
            out_ref[...] = jax.nn.gelu(h)
    return _inv_body


def _run_trunk(h0T, w1t, b1, w2t, b2, wr, wi, wst, bs):
    f32 = jnp.float32
    hT = pl.pallas_call(
        _lift_body, out_shape=jax.ShapeDtypeStruct((HID, G * G * G), f32),
    )(h0T, w1t, b1, w2t, b2)
    for l in range(N_LAYERS):
        sr, si = pl.pallas_call(
            _fwd_body,
            out_shape=[jax.ShapeDtypeStruct((HID, M * M * M), f32)] * 2,
        )(hT, wr[l], wi[l], _C['KrT'], _C['KiT'])
        last = l == N_LAYERS - 1
        out_sh = (G * G * G, HID) if last else (HID, G * G * G)
        hT = pl.pallas_call(
            _make_inv_body(last),
            out_shape=jax.ShapeDtypeStruct(out_sh, f32),
        )(sr, si, hT, wst[l], bs[l], _C['BrT'], _C['BiT'])
    return hT


_QT = 512

def _topk_body(x_ref, yt_ref, idx_ref, mask_ref, cnt_ref):
    x = x_ref[...]
    yt = yt_ref[...]
    xn = jnp.sum(x * x, axis=1, keepdims=True)
    yn = jnp.sum(yt * yt, axis=0, keepdims=True)
    d2 = xn + yn - 2.0 * jnp.dot(x, yt, preferred_element_type=jnp.float32)
    col = lax.broadcasted_iota(jnp.int32, d2.shape, 1)
    big = jnp.int32(1 << 30)
    ids = []
    vals = []
    for _ in range(K_NBR):
        m = jnp.min(d2, axis=1, keepdims=True)
        eq = d2 == m
        am = jnp.min(jnp.where(eq, col, big), axis=1, keepdims=True)
        ids.append(am)
        vals.append(m)
        d2 = jnp.where(col == am, jnp.float32(np.inf), d2)
    idx = jnp.concatenate(ids, axis=1)
    val = jnp.concatenate(vals, axis=1)
    mask = (val <= jnp.float32(RADIUS * RADIUS)).astype(jnp.float32)
    idx_ref[...] = idx
    mask_ref[...] = mask
    cnt_ref[...] = jnp.maximum(jnp.sum(mask, axis=1, keepdims=True), 1.0)


def _run_topk(x_pts, yT):
    n_tile = N_OUT // _QT
    return pl.pallas_call(
        _topk_body,
        grid=(n_tile,),
        in_specs=[
            pl.BlockSpec((_QT, 3), lambda i: (i, 0)),
            pl.BlockSpec((3, G * G * G), lambda i: (0, 0)),
        ],
        out_specs=[
            pl.BlockSpec((_QT, K_NBR), lambda i: (i, 0)),
            pl.BlockSpec((_QT, K_NBR), lambda i: (i, 0)),
            pl.BlockSpec((_QT, 1), lambda i: (i, 0)),
        ],
        out_shape=[
            jax.ShapeDtypeStruct((N_OUT, K_NBR), jnp.int32),
            jax.ShapeDtypeStruct((N_OUT, K_NBR), jnp.float32),
            jax.ShapeDtypeStruct((N_OUT, 1), jnp.float32),
        ],
    )(x_pts, yT)


_D = 48
_B = N_OUT * K_NBR
_CHUNK = 128


def _sc_gather(table, idx2d):
    info = plsc.get_sparse_core_info()
    NC, NS = info.num_cores, info.num_subcores
    NW = NC * NS
    b_per_w = _B // NW
    n_chunk = b_per_w // _CHUNK
    mesh = plsc.VectorSubcoreMesh(core_axis_name="c", subcore_axis_name="s")

    @functools.partial(
        pl.kernel,
        out_type=jax.ShapeDtypeStruct((_B, _D), jnp.float32),
        mesh=mesh,
        compiler_params=pltpu.CompilerParams(use_tc_tiling_on_sc=False),
        scratch_types=[
            pltpu.VMEM((_CHUNK,), jnp.int32),
            pltpu.VMEM((_CHUNK,), jnp.int32),
            pltpu.VMEM((_CHUNK, _D), jnp.float32),
            pltpu.VMEM((_CHUNK, _D), jnp.float32),
            pltpu.SemaphoreType.DMA,
            pltpu.SemaphoreType.DMA,
        ],
    )
    def k(table_hbm, idx_hbm, out_hbm, idx_a, idx_b, rows_a, rows_b,
          sem_a, sem_b):
        wid = lax.axis_index("s") * NC + lax.axis_index("c")
        base = wid * b_per_w
        bufs = [(idx_a, rows_a, sem_a), (idx_b, rows_b, sem_b)]
        cps = [None, None]

        def start(c):
            iv, rv, sm = bufs[c % 2]
            pltpu.sync_copy(idx_hbm.at[wid * n_chunk + c], iv)
            cps[c % 2] = pltpu.async_copy(table_hbm.at[iv], rv, sm)

        start(0)
        for c in range(n_chunk):
            if c + 1 < n_chunk:
                start(c + 1)
            cps[c % 2].wait()
            rv = bufs[c % 2][1]
            pltpu.sync_copy(rv, out_hbm.at[pl.ds(base + c * _CHUNK, _CHUNK)])

    return k(table, idx2d)


_PT = 4096
_QO = _PT // K_NBR


def _mlp_body(g_ref, xi_ref, mk_ref, cnt_ref,
              w1, b1, w2, b2, w3, b3, pw1, pb1, pw2, pb2, out_ref):
    act = jax.nn.gelu
    dot = functools.partial(jnp.dot, preferred_element_type=jnp.float32)
    g = g_ref[...]
    kin = jnp.concatenate([xi_ref[...], g[:, HID:HID + 3]], axis=1)
    k1 = act(dot(kin, w1[...]) + b1[...])
    k2 = act(dot(k1.astype(jnp.bfloat16), w2[...]) + b2[...])
    k3 = dot(k2.astype(jnp.bfloat16), w3[...]) + b3[...]
    contrib = k3 * g[:, :HID] * mk_ref[...]
    pooled = jnp.sum(contrib.reshape(_QO, K_NBR, HID), axis=1)
    feat = pooled / cnt_ref[...]
    p1 = act(dot(feat, pw1[...]) + pb1[...])
    out_ref[...] = dot(p1, pw2[...]) + pb2[...]


def _run_mlp(gathered, xi, maskcol, cnt, w1, b1, w2, b2, w3, b3,
             pw1, pb1, pw2, pb2):
    n_tile = _B // _PT
    const = lambda shape: pl.BlockSpec(shape, lambda i: (0, 0))
    return pl.pallas_call(
        _mlp_body,
        grid=(n_tile,),
        in_specs=[
            pl.BlockSpec((_PT, _D), lambda i: (i, 0)),
            pl.BlockSpec((_PT, 3), lambda i: (i, 0)),
            pl.BlockSpec((_PT, 1), lambda i: (i, 0)),
            pl.BlockSpec((_QO, 1), lambda i: (i, 0)),
            const((6, 512)), const((1, 512)),
            const((512, 256)), const((1, 256)),
            const((256, HID)), const((1, HID)),
            const((HID, 256)), const((1, 256)),
            const((256, 1)), const((1, 1)),
        ],
        out_specs=pl.BlockSpec((_QO, 1), lambda i: (i, 0)),
        out_shape=jax.ShapeDtypeStruct((N_OUT, 1), jnp.float32),
    )(gathered, xi, maskcol, cnt, w1, b1, w2, b2, w3, b3, pw1, pb1, pw2, pb2)


def kernel(in_p, f, out_p, W_lift1, b_lift1, W_lift2, b_lift2, Wspec_r,
           Wspec_i, W_skip, b_skip, G_W1, G_b1, G_W2, G_b2, G_W3, G_b3,
           P_W1, P_b1, P_W2, P_b2):
    y_pts = in_p[0].reshape(-1, 3)
    x_pts = out_p[0]
    h0T = jnp.concatenate([f[0].reshape(-1, 3), y_pts], axis=1).T

    latent = _run_trunk(
        h0T, W_lift1.T, b_lift1[:, None], W_lift2.T, b_lift2[:, None],
        Wspec_r.reshape(N_LAYERS, HID, HID, M * M * M),
        Wspec_i.reshape(N_LAYERS, HID, HID, M * M * M),
        jnp.transpose(W_skip, (0, 2, 1)), b_skip[:, :, None])

    idx, maskv, cnt = _run_topk(x_pts, y_pts.T)

    table = jnp.concatenate(
        [latent, y_pts, jnp.zeros((G * G * G, _D - HID - 3), jnp.float32)], axis=1)
    gathered = _sc_gather(table, idx.reshape(_B // _CHUNK, _CHUNK))

    xi = jnp.broadcast_to(x_pts[:, None, :], (N_OUT, K_NBR, 3)).reshape(_B, 3)
    y = _run_mlp(gathered, xi, maskv.reshape(_B, 1), cnt,
                 G_W1, G_b1[None], G_W2.astype(jnp.bfloat16), G_b2[None],
                 G_W3.astype(jnp.bfloat16), G_b3[None],
                 P_W1, P_b1[None], P_W2, P_b2[None])
    return y[None]


# speedup vs baseline: 1.2495x; 1.2495x over previous
import functools
import numpy as np
import jax
import jax.numpy as jnp
from jax import lax
from jax.experimental import pallas as pl
from jax.experimental.pallas import tpu as pltpu
from jax.experimental.pallas import tpu_sc as plsc

G = 16
M = 8
HID = 32
N_OUT = 8192
K_NBR = 16
RADIUS = 0.1
N_LAYERS = 4


def _dft_consts():
    n = np.arange(G)
    k = np.arange(M)
    F = np.exp(-2j * np.pi * np.outer(k, n) / G) / G
    K3 = np.kron(F, np.kron(F, F))
    E = np.exp(2j * np.pi * np.outer(n, k) / G)
    w = np.ones(M); w[1:] = 2.0
    B3 = np.kron(E, np.kron(E, E * w[None, :]))
    c = {
        'KrT': K3.real.T, 'KiT': K3.imag.T,
        'BrT': B3.T.real, 'BiT': B3.T.imag,
    }
    return {k_: np.asarray(v, np.float32) for k_, v in c.items()}

_C = _dft_consts()


def _lift_body(h0T, w1t, b1, w2t, b2, out_ref):
    act = jax.nn.gelu
    dot = functools.partial(jnp.dot, preferred_element_type=jnp.float32)
    h = act(dot(w1t[...], h0T[...]) + b1[...])
    out_ref[...] = dot(w2t[...], h) + b2[...]


def _fwd_body(hT, wrl, wil, krt, kit, sr_ref, si_ref):
    dot = functools.partial(jnp.dot, preferred_element_type=jnp.float32)
    h = hT[...]
    Xr = dot(h, krt[...])
    Xi = dot(h, kit[...])
    Wrl = wrl[...]
    Wil = wil[...]
    Xr3 = Xr[:, None, :]
    Xi3 = Xi[:, None, :]
    sr_ref[...] = jnp.sum(Xr3 * Wrl - Xi3 * Wil, axis=0)
    si_ref[...] = jnp.sum(Xr3 * Wil + Xi3 * Wrl, axis=0)


def _make_inv_body(last):
    def _inv_body(sr, si, hT, wst_l, bs_l, brt, bit, out_ref):
        dot = functools.partial(jnp.dot, preferred_element_type=jnp.float32)
        xs = dot(sr[...], brt[...]) - dot(si[...], bit[...])
        h = xs + dot(wst_l[...], hT[...]) + bs_l[...]
        if last:
            out_ref[...] = h.T
        else:
            out_ref[...] = jax.nn.gelu(h)
    return _inv_body


def _run_trunk(h0T, w1t, b1, w2t, b2, wr, wi, wst, bs):
    f32 = jnp.float32
    hT = pl.pallas_call(
        _lift_body, out_shape=jax.ShapeDtypeStruct((HID, G * G * G), f32),
    )(h0T, w1t, b1, w2t, b2)
    for l in range(N_LAYERS):
        sr, si = pl.pallas_call(
            _fwd_body,
            out_shape=[jax.ShapeDtypeStruct((HID, M * M * M), f32)] * 2,
        )(hT, wr[l], wi[l], _C['KrT'], _C['KiT'])
        last = l == N_LAYERS - 1
        out_sh = (G * G * G, HID) if last else (HID, G * G * G)
        hT = pl.pallas_call(
            _make_inv_body(last),
            out_shape=jax.ShapeDtypeStruct(out_sh, f32),
        )(sr, si, hT, wst[l], bs[l], _C['BrT'], _C['BiT'])
    return hT


_QT = 512

def _topk_body(x_ref, yt_ref, idx_ref, mask_ref, cnt_ref):
    x = x_ref[...]
    yt = yt_ref[...]
    xn = jnp.sum(x * x, axis=1, keepdims=True)
    yn = jnp.sum(yt * yt, axis=0, keepdims=True)
    d2 = xn + yn - 2.0 * jnp.dot(x, yt, preferred_element_type=jnp.float32)
    d2 = jnp.maximum(d2, 0.0)
    bits = lax.bitcast_convert_type(d2, jnp.int32)
    col = lax.broadcasted_iota(jnp.int32, d2.shape, 1)
    key = (bits & jnp.int32(~4095)) | col
    maxi = jnp.int32(0x7FFFFFFF)
    ids = []
    for _ in range(K_NBR):
        m = jnp.min(key, axis=1, keepdims=True)
        ids.append(m)
        key = jnp.where(key == m, maxi, key)
    mk = jnp.concatenate(ids, axis=1)
    idx_ref[...] = mk & jnp.int32(4095)
    val = lax.bitcast_convert_type(mk & jnp.int32(~4095), jnp.float32)
    mask = (val <= jnp.float32(RADIUS * RADIUS)).astype(jnp.float32)
    mask_ref[...] = mask
    cnt_ref[...] = jnp.maximum(jnp.sum(mask, axis=1, keepdims=True), 1.0)


def _run_topk(x_pts, yT):
    n_tile = N_OUT // _QT
    return pl.pallas_call(
        _topk_body,
        grid=(n_tile,),
        in_specs=[
            pl.BlockSpec((_QT, 3), lambda i: (i, 0)),
            pl.BlockSpec((3, G * G * G), lambda i: (0, 0)),
        ],
        out_specs=[
            pl.BlockSpec((_QT, K_NBR), lambda i: (i, 0)),
            pl.BlockSpec((_QT, K_NBR), lambda i: (i, 0)),
            pl.BlockSpec((_QT, 1), lambda i: (i, 0)),
        ],
        out_shape=[
            jax.ShapeDtypeStruct((N_OUT, K_NBR), jnp.int32),
            jax.ShapeDtypeStruct((N_OUT, K_NBR), jnp.float32),
            jax.ShapeDtypeStruct((N_OUT, 1), jnp.float32),
        ],
    )(x_pts, yT)


_D = 48
_B = N_OUT * K_NBR
_CHUNK = 128


def _sc_gather(table, idx2d):
    info = plsc.get_sparse_core_info()
    NC, NS = info.num_cores, info.num_subcores
    NW = NC * NS
    b_per_w = _B // NW
    n_chunk = b_per_w // _CHUNK
    mesh = plsc.VectorSubcoreMesh(core_axis_name="c", subcore_axis_name="s")

    @functools.partial(
        pl.kernel,
        out_type=jax.ShapeDtypeStruct((_B, _D), jnp.float32),
        mesh=mesh,
        compiler_params=pltpu.CompilerParams(use_tc_tiling_on_sc=False),
        scratch_types=[
            pltpu.VMEM((_CHUNK,), jnp.int32),
            pltpu.VMEM((_CHUNK,), jnp.int32),
            pltpu.VMEM((_CHUNK, _D), jnp.float32),
            pltpu.VMEM((_CHUNK, _D), jnp.float32),
            pltpu.SemaphoreType.DMA,
            pltpu.SemaphoreType.DMA,
        ],
    )
    def k(table_hbm, idx_hbm, out_hbm, idx_a, idx_b, rows_a, rows_b,
          sem_a, sem_b):
        wid = lax.axis_index("s") * NC + lax.axis_index("c")
        base = wid * b_per_w
        bufs = [(idx_a, rows_a, sem_a), (idx_b, rows_b, sem_b)]
        cps = [None, None]

        def start(c):
            iv, rv, sm = bufs[c % 2]
            pltpu.sync_copy(idx_hbm.at[wid * n_chunk + c], iv)
            cps[c % 2] = pltpu.async_copy(table_hbm.at[iv], rv, sm)

        start(0)
        for c in range(n_chunk):
            if c + 1 < n_chunk:
                start(c + 1)
            cps[c % 2].wait()
            rv = bufs[c % 2][1]
            pltpu.sync_copy(rv, out_hbm.at[pl.ds(base + c * _CHUNK, _CHUNK)])

    return k(table, idx2d)


_PT = 4096
_QO = _PT // K_NBR


def _mlp_body(g_ref, xi_ref, mk_ref, cnt_ref,
              w1, b1, w2, b2, w3, b3, pw1, pb1, pw2, pb2, out_ref):
    act = jax.nn.gelu
    dot = functools.partial(jnp.dot, preferred_element_type=jnp.float32)
    g = g_ref[...]
    kin = jnp.concatenate([xi_ref[...], g[:, HID:HID + 3]], axis=1)
    k1 = act(dot(kin, w1[...]) + b1[...])
    k2 = act(dot(k1.astype(jnp.bfloat16), w2[...]) + b2[...])
    k3 = dot(k2.astype(jnp.bfloat16), w3[...]) + b3[...]
    contrib = k3 * g[:, :HID] * mk_ref[...]
    pooled = jnp.sum(contrib.reshape(_QO, K_NBR, HID), axis=1)
    feat = pooled / cnt_ref[...]
    p1 = act(dot(feat, pw1[...]) + pb1[...])
    out_ref[...] = dot(p1, pw2[...]) + pb2[...]


def _run_mlp(gathered, xi, maskcol, cnt, w1, b1, w2, b2, w3, b3,
             pw1, pb1, pw2, pb2):
    n_tile = _B // _PT
    const = lambda shape: pl.BlockSpec(shape, lambda i: (0, 0))
    return pl.pallas_call(
        _mlp_body,
        grid=(n_tile,),
        in_specs=[
            pl.BlockSpec((_PT, _D), lambda i: (i, 0)),
            pl.BlockSpec((_PT, 3), lambda i: (i, 0)),
            pl.BlockSpec((_PT, 1), lambda i: (i, 0)),
            pl.BlockSpec((_QO, 1), lambda i: (i, 0)),
            const((6, 512)), const((1, 512)),
            const((512, 256)), const((1, 256)),
            const((256, HID)), const((1, HID)),
            const((HID, 256)), const((1, 256)),
            const((256, 1)), const((1, 1)),
        ],
        out_specs=pl.BlockSpec((_QO, 1), lambda i: (i, 0)),
        out_shape=jax.ShapeDtypeStruct((N_OUT, 1), jnp.float32),
    )(gathered, xi, maskcol, cnt, w1, b1, w2, b2, w3, b3, pw1, pb1, pw2, pb2)


def kernel(in_p, f, out_p, W_lift1, b_lift1, W_lift2, b_lift2, Wspec_r,
           Wspec_i, W_skip, b_skip, G_W1, G_b1, G_W2, G_b2, G_W3, G_b3,
           P_W1, P_b1, P_W2, P_b2):
    y_pts = in_p[0].reshape(-1, 3)
    x_pts = out_p[0]
    h0T = jnp.concatenate([f[0].reshape(-1, 3), y_pts], axis=1).T

    latent = _run_trunk(
        h0T, W_lift1.T, b_lift1[:, None], W_lift2.T, b_lift2[:, None],
        Wspec_r.reshape(N_LAYERS, HID, HID, M * M * M),
        Wspec_i.reshape(N_LAYERS, HID, HID, M * M * M),
        jnp.transpose(W_skip, (0, 2, 1)), b_skip[:, :, None])

    idx, maskv, cnt = _run_topk(x_pts, y_pts.T)

    table = jnp.concatenate(
        [latent, y_pts, jnp.zeros((G * G * G, _D - HID - 3), jnp.float32)], axis=1)
    gathered = _sc_gather(table, idx.reshape(_B // _CHUNK, _CHUNK))

    xi = jnp.broadcast_to(x_pts[:, None, :], (N_OUT, K_NBR, 3)).reshape(_B, 3)
    y = _run_mlp(gathered, xi, maskv.reshape(_B, 1), cnt,
                 G_W1, G_b1[None], G_W2.astype(jnp.bfloat16), G_b2[None],
                 G_W3.astype(jnp.bfloat16), G_b3[None],
                 P_W1, P_b1[None], P_W2, P_b2[None])
    return y[None]


# speedup vs baseline: 1.4094x; 1.1280x over previous
import functools
import numpy as np
import jax
import jax.numpy as jnp
from jax import lax
from jax.experimental import pallas as pl
from jax.experimental.pallas import tpu as pltpu
from jax.experimental.pallas import tpu_sc as plsc

G = 16
M = 8
HID = 32
N_OUT = 8192
K_NBR = 16
RADIUS = 0.1
N_LAYERS = 4


def _dft_consts():
    n = np.arange(G)
    k = np.arange(M)
    F = np.exp(-2j * np.pi * np.outer(k, n) / G) / G
    K3 = np.kron(F, np.kron(F, F))
    E = np.exp(2j * np.pi * np.outer(n, k) / G)
    w = np.ones(M); w[1:] = 2.0
    B3 = np.kron(E, np.kron(E, E * w[None, :]))
    c = {
        'KrT': K3.real.T, 'KiT': K3.imag.T,
        'BrT': B3.T.real, 'BiT': B3.T.imag,
    }
    return {k_: np.asarray(v, np.float32) for k_, v in c.items()}

_C = _dft_consts()


def _lift_body(h0T, w1t, b1, w2t, b2, out_ref):
    act = jax.nn.gelu
    dot = functools.partial(jnp.dot, preferred_element_type=jnp.float32)
    h = act(dot(w1t[...], h0T[...]) + b1[...])
    out_ref[...] = dot(w2t[...], h) + b2[...]


def _fno_body(hT0, wrl, wil, wst_l, bs_l, krt, kit, brt, bit,
              out_ref, h_s):
    dot = functools.partial(jnp.dot, preferred_element_type=jnp.float32)
    l = pl.program_id(0)

    @pl.when(l == 0)
    def _():
        h_s[...] = hT0[...]

    h = h_s[...]
    Xr = dot(h, krt[...])
    Xi = dot(h, kit[...])
    Wrl = wrl[0]
    Wil = wil[0]
    Xr3 = Xr[:, None, :]
    Xi3 = Xi[:, None, :]
    Sr = jnp.sum(Xr3 * Wrl - Xi3 * Wil, axis=0)
    Si = jnp.sum(Xr3 * Wil + Xi3 * Wrl, axis=0)
    xs = dot(Sr, brt[...]) - dot(Si, bit[...])
    hn = xs + dot(wst_l[0], h) + bs_l[0]

    @pl.when(l < N_LAYERS - 1)
    def _():
        h_s[...] = jax.nn.gelu(hn)

    @pl.when(l == N_LAYERS - 1)
    def _():
        out_ref[...] = hn.T


def _run_trunk(h0T, w1t, b1, w2t, b2, wr, wi, wst, bs):
    f32 = jnp.float32
    hT = pl.pallas_call(
        _lift_body, out_shape=jax.ShapeDtypeStruct((HID, G * G * G), f32),
    )(h0T, w1t, b1, w2t, b2)
    NG = G * G * G
    NM = M * M * M
    const = lambda shape: pl.BlockSpec(shape, lambda i: (0,) * len(shape))
    return pl.pallas_call(
        _fno_body,
        grid=(N_LAYERS,),
        in_specs=[
            const((HID, NG)),
            pl.BlockSpec((1, HID, HID, NM), lambda i: (i, 0, 0, 0)),
            pl.BlockSpec((1, HID, HID, NM), lambda i: (i, 0, 0, 0)),
            pl.BlockSpec((1, HID, HID), lambda i: (i, 0, 0)),
            pl.BlockSpec((1, HID, 1), lambda i: (i, 0, 0)),
            const((NG, NM)), const((NG, NM)),
            const((NM, NG)), const((NM, NG)),
        ],
        out_specs=pl.BlockSpec((NG, HID), lambda i: (0, 0)),
        out_shape=jax.ShapeDtypeStruct((NG, HID), f32),
        scratch_shapes=[pltpu.VMEM((HID, NG), f32)],
    )(hT, wr, wi, wst, bs, _C['KrT'], _C['KiT'], _C['BrT'], _C['BiT'])


_QT = 512

def _topk_body(x_ref, yt_ref, idx_ref, mask_ref, cnt_ref):
    x = x_ref[...]
    yt = yt_ref[...]
    xn = jnp.sum(x * x, axis=1, keepdims=True)
    yn = jnp.sum(yt * yt, axis=0, keepdims=True)
    d2 = xn + yn - 2.0 * jnp.dot(x, yt, preferred_element_type=jnp.float32)
    d2 = jnp.maximum(d2, 0.0)
    bits = lax.bitcast_convert_type(d2, jnp.int32)
    col = lax.broadcasted_iota(jnp.int32, d2.shape, 1)
    key = (bits & jnp.int32(~4095)) | col
    maxi = jnp.int32(0x7FFFFFFF)
    ids = []
    for _ in range(K_NBR):
        m = jnp.min(key, axis=1, keepdims=True)
        ids.append(m)
        key = jnp.where(key == m, maxi, key)
    mk = jnp.concatenate(ids, axis=1)
    idx_ref[...] = mk & jnp.int32(4095)
    val = lax.bitcast_convert_type(mk & jnp.int32(~4095), jnp.float32)
    mask = (val <= jnp.float32(RADIUS * RADIUS)).astype(jnp.float32)
    mask_ref[...] = mask
    cnt_ref[...] = jnp.maximum(jnp.sum(mask, axis=1, keepdims=True), 1.0)


def _run_topk(x_pts, yT):
    n_tile = N_OUT // _QT
    return pl.pallas_call(
        _topk_body,
        grid=(n_tile,),
        in_specs=[
            pl.BlockSpec((_QT, 3), lambda i: (i, 0)),
            pl.BlockSpec((3, G * G * G), lambda i: (0, 0)),
        ],
        out_specs=[
            pl.BlockSpec((_QT, K_NBR), lambda i: (i, 0)),
            pl.BlockSpec((_QT, K_NBR), lambda i: (i, 0)),
            pl.BlockSpec((_QT, 1), lambda i: (i, 0)),
        ],
        out_shape=[
            jax.ShapeDtypeStruct((N_OUT, K_NBR), jnp.int32),
            jax.ShapeDtypeStruct((N_OUT, K_NBR), jnp.float32),
            jax.ShapeDtypeStruct((N_OUT, 1), jnp.float32),
        ],
    )(x_pts, yT)


_D = 48
_B = N_OUT * K_NBR
_CHUNK = 128


def _sc_gather(table, idx2d):
    info = plsc.get_sparse_core_info()
    NC, NS = info.num_cores, info.num_subcores
    NW = NC * NS
    b_per_w = _B // NW
    n_chunk = b_per_w // _CHUNK
    mesh = plsc.VectorSubcoreMesh(core_axis_name="c", subcore_axis_name="s")

    @functools.partial(
        pl.kernel,
        out_type=jax.ShapeDtypeStruct((_B, _D), jnp.float32),
        mesh=mesh,
        compiler_params=pltpu.CompilerParams(use_tc_tiling_on_sc=False),
        scratch_types=[
            pltpu.VMEM((_CHUNK,), jnp.int32),
            pltpu.VMEM((_CHUNK,), jnp.int32),
            pltpu.VMEM((_CHUNK, _D), jnp.float32),
            pltpu.VMEM((_CHUNK, _D), jnp.float32),
            pltpu.SemaphoreType.DMA,
            pltpu.SemaphoreType.DMA,
        ],
    )
    def k(table_hbm, idx_hbm, out_hbm, idx_a, idx_b, rows_a, rows_b,
          sem_a, sem_b):
        wid = lax.axis_index("s") * NC + lax.axis_index("c")
        base = wid * b_per_w
        bufs = [(idx_a, rows_a, sem_a), (idx_b, rows_b, sem_b)]
        cps = [None, None]

        def start(c):
            iv, rv, sm = bufs[c % 2]
            pltpu.sync_copy(idx_hbm.at[wid * n_chunk + c], iv)
            cps[c % 2] = pltpu.async_copy(table_hbm.at[iv], rv, sm)

        start(0)
        for c in range(n_chunk):
            if c + 1 < n_chunk:
                start(c + 1)
            cps[c % 2].wait()
            rv = bufs[c % 2][1]
            pltpu.sync_copy(rv, out_hbm.at[pl.ds(base + c * _CHUNK, _CHUNK)])

    return k(table, idx2d)


_PT = 4096
_QO = _PT // K_NBR


def _mlp_body(g_ref, xi_ref, mk_ref, cnt_ref,
              w1, b1, w2, b2, w3, b3, pw1, pb1, pw2, pb2, out_ref):
    act = jax.nn.gelu
    dot = functools.partial(jnp.dot, preferred_element_type=jnp.float32)
    g = g_ref[...]
    kin = jnp.concatenate([xi_ref[...], g[:, HID:HID + 3]], axis=1)
    k1 = act(dot(kin, w1[...]) + b1[...])
    k2 = act(dot(k1.astype(jnp.bfloat16), w2[...]) + b2[...])
    k3 = dot(k2.astype(jnp.bfloat16), w3[...]) + b3[...]
    contrib = k3 * g[:, :HID] * mk_ref[...]
    pooled = jnp.sum(contrib.reshape(_QO, K_NBR, HID), axis=1)
    feat = pooled / cnt_ref[...]
    p1 = act(dot(feat, pw1[...]) + pb1[...])
    out_ref[...] = dot(p1, pw2[...]) + pb2[...]


def _run_mlp(gathered, xi, maskcol, cnt, w1, b1, w2, b2, w3, b3,
             pw1, pb1, pw2, pb2):
    n_tile = _B // _PT
    const = lambda shape: pl.BlockSpec(shape, lambda i: (0, 0))
    return pl.pallas_call(
        _mlp_body,
        grid=(n_tile,),
        in_specs=[
            pl.BlockSpec((_PT, _D), lambda i: (i, 0)),
            pl.BlockSpec((_PT, 3), lambda i: (i, 0)),
            pl.BlockSpec((_PT, 1), lambda i: (i, 0)),
            pl.BlockSpec((_QO, 1), lambda i: (i, 0)),
            const((6, 512)), const((1, 512)),
            const((512, 256)), const((1, 256)),
            const((256, HID)), const((1, HID)),
            const((HID, 256)), const((1, 256)),
            const((256, 1)), const((1, 1)),
        ],
        out_specs=pl.BlockSpec((_QO, 1), lambda i: (i, 0)),
        out_shape=jax.ShapeDtypeStruct((N_OUT, 1), jnp.float32),
    )(gathered, xi, maskcol, cnt, w1, b1, w2, b2, w3, b3, pw1, pb1, pw2, pb2)


def kernel(in_p, f, out_p, W_lift1, b_lift1, W_lift2, b_lift2, Wspec_r,
           Wspec_i, W_skip, b_skip, G_W1, G_b1, G_W2, G_b2, G_W3, G_b3,
           P_W1, P_b1, P_W2, P_b2):
    y_pts = in_p[0].reshape(-1, 3)
    x_pts = out_p[0]
    h0T = jnp.concatenate([f[0].reshape(-1, 3), y_pts], axis=1).T

    latent = _run_trunk(
        h0T, W_lift1.T, b_lift1[:, None], W_lift2.T, b_lift2[:, None],
        Wspec_r.reshape(N_LAYERS, HID, HID, M * M * M),
        Wspec_i.reshape(N_LAYERS, HID, HID, M * M * M),
        jnp.transpose(W_skip, (0, 2, 1)), b_skip[:, :, None])

    idx, maskv, cnt = _run_topk(x_pts, y_pts.T)

    table = jnp.concatenate(
        [latent, y_pts, jnp.zeros((G * G * G, _D - HID - 3), jnp.float32)], axis=1)
    gathered = _sc_gather(table, idx.reshape(_B // _CHUNK, _CHUNK))

    xi = jnp.broadcast_to(x_pts[:, None, :], (N_OUT, K_NBR, 3)).reshape(_B, 3)
    y = _run_mlp(gathered, xi, maskv.reshape(_B, 1), cnt,
                 G_W1, G_b1[None], G_W2.astype(jnp.bfloat16), G_b2[None],
                 G_W3.astype(jnp.bfloat16), G_b3[None],
                 P_W1, P_b1[None], P_W2, P_b2[None])
    return y[None]
